# baseline (speedup 1.0000x reference)
import functools

import jax
import jax.numpy as jnp
from jax import lax
from jax.experimental import pallas as pl
from jax.experimental.pallas import tpu as pltpu
from jax.experimental.pallas import tpu_sc as plsc

D0, D1, D2 = 480, 360, 32
S0, S1 = D1 * D2, D2
N = 200000
C = 32

NPAD = 204800
ZROWS = NPAD - N
TSIZE = D0 * D1 * D2
SNT_BASE = TSIZE + 8192
TTOT = 5767168

NW_B = 32
PW_B = NPAD // NW_B
TOFF = 4096
HHALF = TTOT // 2
TRASH1_BASE = TOFF + TSIZE + 8192 + 16384
PW_A = NPAD // 16
TW_A = TTOT // 32
MSET_CH = 16384
NMSET = TW_A // MSET_CH

NR = NPAD // 4
BLK = 512
NBLK = NR // BLK
NVALID4 = N // 4


def _keys16(c0, c1, c2):
    return c0 * S0 + c1 * S1 + c2


def _table_build_body(c0h, c1h, c2h, table_h, mset, c0v, c1v, c2v,
                      keyb0, keyb1, valb0, valb1, semm, sems0, sems1):
    ci = lax.axis_index("c")
    si = lax.axis_index("s")
    keybs = (keyb0, keyb1)
    valbs = (valb0, valb1)
    semss = (sems0, sems1)

    def fill(i, carry):
        mset[pl.ds(i * 16, 16)] = jnp.full((16,), -1, jnp.int32)
        return carry

    lax.fori_loop(0, MSET_CH // 16, fill, 0)
    mbase = ci * HHALF + si * TW_A

    def memset(i, carry):
        pltpu.async_copy(mset, table_h.at[pl.ds(mbase + i * MSET_CH,
                                                MSET_CH)], semm)
        return carry

    lax.fori_loop(0, NMSET, memset, 0)

    def memdrain(i, carry):
        pltpu.make_async_copy(mset, table_h.at[pl.ds(mbase, MSET_CH)],
                              semm).wait()
        return carry

    lax.fori_loop(0, NMSET, memdrain, 0)
    plsc.subcore_barrier()

    base = si * PW_A
    pltpu.sync_copy(c0h.at[pl.ds(base, PW_A)], c0v)
    pltpu.sync_copy(c1h.at[pl.ds(base, PW_A)], c1v)
    pltpu.sync_copy(c2h.at[pl.ds(base, PW_A)], c2v)
    lo = ci * HHALF
    tb = ci * TRASH1_BASE

    def scatter(t2, carry):
        for q in range(2):
            c = t2 * 2 + q

            @pl.when(c >= 2)
            def _():
                pltpu.make_async_copy(valbs[q], table_h.at[keybs[q]],
                                      semss[q]).wait()

            for g in range(8):
                off = c * 128 + g * 16
                slot = TOFF + _keys16(c0v[pl.ds(off, 16)],
                                      c1v[pl.ds(off, 16)],
                                      c2v[pl.ds(off, 16)])
                ig = base + off + lax.broadcasted_iota(jnp.int32, (16,), 0)
                mine = (slot >= lo) & (slot < lo + HHALF)
                trash = tb + ((ig & 1023) << 2)
                keybs[q][pl.ds(g * 16, 16)] = jnp.where(mine, slot, trash)
                valbs[q][pl.ds(g * 16, 16)] = ig
            pltpu.async_copy(valbs[q], table_h.at[keybs[q]], semss[q])
        return carry

    lax.fori_loop(0, PW_A // 256, scatter, 0)
    for q in range(2):
        pltpu.make_async_copy(valbs[q], table_h.at[keybs[q]], semss[q]).wait()


def _build_table(c0f, c1f, c2f):
    mesh = plsc.VectorSubcoreMesh(core_axis_name="c", subcore_axis_name="s")
    return pl.kernel(
        _table_build_body,
        out_type=jax.ShapeDtypeStruct((TTOT,), jnp.int32),
        mesh=mesh,
        compiler_params=pltpu.CompilerParams(use_tc_tiling_on_sc=False),
        scratch_types=[
            pltpu.VMEM((MSET_CH,), jnp.int32),
            pltpu.VMEM((PW_A,), jnp.int32),
            pltpu.VMEM((PW_A,), jnp.int32),
            pltpu.VMEM((PW_A,), jnp.int32),
            pltpu.VMEM((128,), jnp.int32),
            pltpu.VMEM((128,), jnp.int32),
            pltpu.VMEM((128,), jnp.int32),
            pltpu.VMEM((128,), jnp.int32),
            pltpu.SemaphoreType.DMA,
            pltpu.SemaphoreType.DMA,
            pltpu.SemaphoreType.DMA,
        ],
    )(c0f, c1f, c2f)


_TAPS = ((0, -1), (0, 1), (1, -1), (1, 1), (2, -1), (2, 1))


NCH = PW_B // 128


def _gather_body(c0h, c1h, c2h, table_h, featsp_h, o0, o1, o2, o3, o4, o5,
                 c0v, c1v, c2v, nk, jb, zb, rows,
                 sl0, sl1, sg0, sg1, sw0, sw1):
    outs = (o0, o1, o2, o3, o4, o5)
    sls = (sl0, sl1)
    sgs = (sg0, sg1)
    sws = (sw0, sw1)
    wid = lax.axis_index("s") * 2 + lax.axis_index("c")
    base = wid * PW_B
    pltpu.sync_copy(c0h.at[pl.ds(base, PW_B)], c0v)
    pltpu.sync_copy(c1h.at[pl.ds(base, PW_B)], c1v)
    pltpu.sync_copy(c2h.at[pl.ds(base, PW_B)], c2v)

    def lk_copy(p, s, start=True):
        cp = (pltpu.async_copy if start else pltpu.make_async_copy)
        return cp(table_h.at[nk.at[p, s]], jb.at[p, s], sls[p])

    def g_copy(p, s, start=True):
        cp = (pltpu.async_copy if start else pltpu.make_async_copy)
        return cp(featsp_h.at[nk.at[p, s]], rows.at[p, s], sgs[p])

    def w_copy(p, s, c, start=True):
        cp = (pltpu.async_copy if start else pltpu.make_async_copy)
        return cp(rows.at[p, s], outs[s].at[pl.ds(base + c * 128, 128), :],
                  sws[p])

    def step(t2, _):
        for q in range(2):
            t = t2 * 2 + q
            @pl.when((t >= 2) & (t <= NCH + 1))
            def _():
                for s in range(6):
                    g_copy(q, s, start=False).wait()
                for s in range(6):
                    w_copy(q, s, t - 2)

            @pl.when(t < NCH)
            def _():
                for g in range(8):
                    off = t * 128 + g * 16
                    c0 = c0v[pl.ds(off, 16)]
                    c1 = c1v[pl.ds(off, 16)]
                    c2 = c2v[pl.ds(off, 16)]
                    cs = (c0, c1, c2)
                    dims = (D0, D1, D2)
                    strides = (S0, S1, 1)
                    key = _keys16(c0, c1, c2)
                    ig = base + off + lax.broadcasted_iota(jnp.int32,
                                                           (16,), 0)
                    zb[q, pl.ds(g * 16, 16)] = N + lax.rem(ig, ZROWS)
                    snt = TOFF + SNT_BASE + ((ig & 1023) << 4)
                    for s, (ax, d) in enumerate(_TAPS):
                        ca = cs[ax] + d
                        inb = (ca >= 0) & (ca < dims[ax])
                        nk[q, s, pl.ds(g * 16, 16)] = jnp.where(
                            inb, TOFF + key + d * strides[ax], snt)
                for s in range(6):
                    lk_copy(q, s)

            p = 1 - q

            @pl.when((t >= 1) & (t <= NCH))
            def _():
                for s in range(6):
                    lk_copy(p, s, start=False).wait()
                for s in range(6):
                    for g in range(8):
                        j = jb[p, s, pl.ds(g * 16, 16)]
                        nk[p, s, pl.ds(g * 16, 16)] = jnp.where(
                            j < 0, zb[p, pl.ds(g * 16, 16)], j)

                @pl.when(t >= 3)
                def _():
                    for s in range(6):
                        w_copy(p, s, 0, start=False).wait()

                for s in range(6):
                    g_copy(p, s)
        return 0

    lax.fori_loop(0, (NCH + 2) // 2, step, 0)
    for p in range(2):
        for s in range(6):
            w_copy(p, s, 0, start=False).wait()


def _gather_taps(c0f, c1f, c2f, table, feats_p):
    mesh = plsc.VectorSubcoreMesh(core_axis_name="c", subcore_axis_name="s")
    out = jax.ShapeDtypeStruct((NPAD, C), jnp.float32)
    return pl.kernel(
        _gather_body,
        out_type=(out,) * 6,
        mesh=mesh,
        compiler_params=pltpu.CompilerParams(use_tc_tiling_on_sc=False),
        scratch_types=[
            pltpu.VMEM((PW_B,), jnp.int32),
            pltpu.VMEM((PW_B,), jnp.int32),
            pltpu.VMEM((PW_B,), jnp.int32),
            pltpu.VMEM((2, 6, 128), jnp.int32),
            pltpu.VMEM((2, 6, 128), jnp.int32),
            pltpu.VMEM((2, 128), jnp.int32),
            pltpu.VMEM((2, 6, 128, C), jnp.float32),
            pltpu.SemaphoreType.DMA,
            pltpu.SemaphoreType.DMA,
            pltpu.SemaphoreType.DMA,
            pltpu.SemaphoreType.DMA,
            pltpu.SemaphoreType.DMA,
            pltpu.SemaphoreType.DMA,
        ],
    )(c0f, c1f, c2f, table, feats_p)


def _conv_stats_kernel(f_ref, g0, g1, g2, g3, g4, g5, w1, w2, w3,
                       r1, r2, r3, s1, s2, s3, q1, q2, q3,
                       acc_s, acc_q):
    i = pl.program_id(0)

    @pl.when(i == 0)
    def _():
        acc_s[...] = jnp.zeros_like(acc_s)
        acc_q[...] = jnp.zeros_like(acc_q)

    gms = (g0, g2, g4)
    gps = (g1, g3, g5)
    ws = (w1, w2, w3)
    rs = (r1, r2, r3)
    f = f_ref[...]
    row = lax.broadcasted_iota(jnp.int32, (BLK, 1), 0) + i * BLK
    validf = jnp.where(row < NVALID4, 1.0, 0.0).astype(jnp.float32)
    for a in range(3):
        w = ws[a]
        raw = (jnp.dot(gms[a][...], w[0], preferred_element_type=jnp.float32)
               + jnp.dot(f, w[1], preferred_element_type=jnp.float32)
               + jnp.dot(gps[a][...], w[2], preferred_element_type=jnp.float32))
        rs[a][...] = raw
        rm = raw * validf
        acc_s[:, a * 128:(a + 1) * 128] += jnp.sum(
            rm.reshape(BLK // 8, 8, 128), axis=0)
        acc_q[:, a * 128:(a + 1) * 128] += jnp.sum(
            (rm * rm).reshape(BLK // 8, 8, 128), axis=0)

    @pl.when(i == NBLK - 1)
    def _():
        s1[...] = acc_s[:, 0:128]
        s2[...] = acc_s[:, 128:256]
        s3[...] = acc_s[:, 256:384]
        q1[...] = acc_q[:, 0:128]
        q2[...] = acc_q[:, 128:256]
        q3[...] = acc_q[:, 256:384]


def _conv_stats(feats4, gath4, wbd):
    blk = pl.BlockSpec((BLK, 128), lambda i: (i, 0))
    wspec = pl.BlockSpec((3, 128, 128), lambda i: (0, 0, 0))
    sspec = pl.BlockSpec((8, 128), lambda i: (0, 0))
    big = jax.ShapeDtypeStruct((NR, 128), jnp.float32)
    small = jax.ShapeDtypeStruct((8, 128), jnp.float32)
    return pl.pallas_call(
        _conv_stats_kernel,
        grid=(NBLK,),
        in_specs=[blk] * 7 + [wspec] * 3,
        out_specs=[blk] * 3 + [sspec] * 6,
        out_shape=[big] * 3 + [small] * 6,
        scratch_shapes=[pltpu.VMEM((8, 384), jnp.float32),
                        pltpu.VMEM((8, 384), jnp.float32)],
    )(feats4, *gath4, *wbd)


def _apply_kernel(f_ref, r1, r2, r3, sc_ref, out_ref):
    f = f_ref[...]
    rs = (r1, r2, r3)
    tot = jnp.zeros((BLK, 128), jnp.float32)
    for a in range(3):
        x = rs[a][...] * sc_ref[a:a + 1, :] + sc_ref[a + 3:a + 4, :]
        tot = tot + 1.0 / (1.0 + jnp.exp(-x))
    out_ref[...] = tot * f


def _apply(feats4, raws, scsh):
    blk = pl.BlockSpec((BLK, 128), lambda i: (i, 0))
    sspec = pl.BlockSpec((8, 128), lambda i: (0, 0))
    return pl.pallas_call(
        _apply_kernel,
        grid=(NBLK,),
        in_specs=[blk] * 4 + [sspec],
        out_specs=blk,
        out_shape=jax.ShapeDtypeStruct((NR, 128), jnp.float32),
    )(feats4, *raws, scsh)


def kernel(feats, coords, W1, W2, W3, g1, b1, g2, b2, g3, b3):
    r = jnp.arange(ZROWS, dtype=jnp.int32)
    c0f = jnp.concatenate([coords[:, 0], jnp.full((ZROWS,), D0, jnp.int32)])
    c1f = jnp.concatenate([coords[:, 1], r // 32])
    c2f = jnp.concatenate([coords[:, 2], r % 32])
    feats_p = jnp.zeros((NPAD, C), jnp.float32).at[:N].set(feats)

    table = _build_table(c0f, c1f, c2f)
    gath = _gather_taps(c0f, c1f, c2f, table, feats_p)

    feats4 = feats_p.reshape(NR, 128)
    gath4 = [g.reshape(NR, 128) for g in gath]

    eye4 = jnp.eye(4, dtype=jnp.float32)
    wbd = [jnp.einsum("kab,ij->kiajb", W, eye4).reshape(3, 128, 128)
           for W in (W1, W2, W3)]
    res = _conv_stats(feats4, gath4, wbd)
    raws = res[0:3]
    sums = res[3:6]
    sqs = res[6:9]

    gs = (g1, g2, g3)
    bs = (b1, b2, b3)
    scsh = []
    for a in range(3):
        s = jnp.sum(sums[a], axis=0).reshape(4, 32).sum(axis=0)
        q = jnp.sum(sqs[a], axis=0).reshape(4, 32).sum(axis=0)
        mean = s / N
        var = q / N - mean * mean
        scale = gs[a] / jnp.sqrt(var + 1e-5)
        shift = bs[a] - mean * scale
        scsh.append(jnp.tile(scale, 4))
        scsh.append(jnp.tile(shift, 4))
    scsh = jnp.stack(scsh[0::2] + scsh[1::2] + [jnp.zeros(128)] * 2)
    scsh = scsh.astype(jnp.float32)

    out4 = _apply(feats4, raws, scsh)
    return out4.reshape(NPAD, C)[:N]


# speedup vs baseline: 1.4858x; 1.4858x over previous
import functools

import jax
import jax.numpy as jnp
from jax import lax
from jax.experimental import pallas as pl
from jax.experimental.pallas import tpu as pltpu
from jax.experimental.pallas import tpu_sc as plsc

D0, D1, D2 = 480, 360, 32
S0, S1 = D1 * D2, D2
N = 200000
C = 32

NPAD = 204800
ZROWS = NPAD - N
TSIZE = D0 * D1 * D2
SNT_BASE = TSIZE + 8192
TTOT = 5767168

NW_B = 32
PW_B = NPAD // NW_B
TOFF = 65536
HHALF = TTOT // 2
TRASH1_BASE = TOFF + TSIZE + 8192 + 16384
PW_A = NPAD // 16
TW_A = TTOT // 32
MSET_CH = 16384
NMSET = TW_A // MSET_CH

NR = NPAD // 4
BLK = 512
NBLK = NR // BLK
NVALID4 = N // 4


def _keys16(c0, c1, c2):
    return c0 * S0 + c1 * S1 + c2


def _table_build_body(c0h, c1h, c2h, table_h, mset, c0v, c1v, c2v,
                      keyb0, keyb1, valb0, valb1, semm, sems0, sems1):
    ci = lax.axis_index("c")
    si = lax.axis_index("s")
    keybs = (keyb0, keyb1)
    valbs = (valb0, valb1)
    semss = (sems0, sems1)

    def fill(i, carry):
        mset[pl.ds(i * 16, 16)] = jnp.full((16,), -1, jnp.int32)
        return carry

    lax.fori_loop(0, MSET_CH // 16, fill, 0)
    mbase = ci * HHALF + si * TW_A

    def memset(i, carry):
        pltpu.async_copy(mset, table_h.at[pl.ds(mbase + i * MSET_CH,
                                                MSET_CH)], semm)
        return carry

    lax.fori_loop(0, NMSET, memset, 0)

    def memdrain(i, carry):
        pltpu.make_async_copy(mset, table_h.at[pl.ds(mbase, MSET_CH)],
                              semm).wait()
        return carry

    lax.fori_loop(0, NMSET, memdrain, 0)
    plsc.subcore_barrier()

    base = si * PW_A
    pltpu.sync_copy(c0h.at[pl.ds(base, PW_A)], c0v)
    pltpu.sync_copy(c1h.at[pl.ds(base, PW_A)], c1v)
    pltpu.sync_copy(c2h.at[pl.ds(base, PW_A)], c2v)
    lo = ci * HHALF
    tb = ci * TRASH1_BASE

    def scatter(t2, carry):
        for q in range(2):
            c = t2 * 2 + q

            @pl.when(c >= 2)
            def _():
                pltpu.make_async_copy(valbs[q], table_h.at[keybs[q]],
                                      semss[q]).wait()

            for g in range(8):
                off = c * 128 + g * 16
                slot = TOFF + _keys16(c0v[pl.ds(off, 16)],
                                      c1v[pl.ds(off, 16)],
                                      c2v[pl.ds(off, 16)])
                ig = base + off + lax.broadcasted_iota(jnp.int32, (16,), 0)
                mine = (slot >= lo) & (slot < lo + HHALF)
                trash = tb + ((ig & 4095) << 4)
                keybs[q][pl.ds(g * 16, 16)] = jnp.where(mine, slot, trash)
                valbs[q][pl.ds(g * 16, 16)] = ig
            pltpu.async_copy(valbs[q], table_h.at[keybs[q]], semss[q])
        return carry

    lax.fori_loop(0, PW_A // 256, scatter, 0)
    for q in range(2):
        pltpu.make_async_copy(valbs[q], table_h.at[keybs[q]], semss[q]).wait()


def _build_table(c0f, c1f, c2f):
    mesh = plsc.VectorSubcoreMesh(core_axis_name="c", subcore_axis_name="s")
    return pl.kernel(
        _table_build_body,
        out_type=jax.ShapeDtypeStruct((TTOT,), jnp.int32),
        mesh=mesh,
        compiler_params=pltpu.CompilerParams(use_tc_tiling_on_sc=False),
        scratch_types=[
            pltpu.VMEM((MSET_CH,), jnp.int32),
            pltpu.VMEM((PW_A,), jnp.int32),
            pltpu.VMEM((PW_A,), jnp.int32),
            pltpu.VMEM((PW_A,), jnp.int32),
            pltpu.VMEM((128,), jnp.int32),
            pltpu.VMEM((128,), jnp.int32),
            pltpu.VMEM((128,), jnp.int32),
            pltpu.VMEM((128,), jnp.int32),
            pltpu.SemaphoreType.DMA,
            pltpu.SemaphoreType.DMA,
            pltpu.SemaphoreType.DMA,
        ],
    )(c0f, c1f, c2f)


_TAPS = ((0, -1), (0, 1), (1, -1), (1, 1), (2, -1), (2, 1))


NCH = PW_B // 128


def _gather_body(c0h, c1h, c2h, table_h, featsp_h, o0, o1, o2, o3, o4, o5,
                 c0v, c1v, c2v, nk, jb, zb, rows,
                 sl0, sl1, sg0, sg1, sw0, sw1):
    outs = (o0, o1, o2, o3, o4, o5)
    sls = (sl0, sl1)
    sgs = (sg0, sg1)
    sws = (sw0, sw1)
    wid = lax.axis_index("s") * 2 + lax.axis_index("c")
    base = wid * PW_B
    pltpu.sync_copy(c0h.at[pl.ds(base, PW_B)], c0v)
    pltpu.sync_copy(c1h.at[pl.ds(base, PW_B)], c1v)
    pltpu.sync_copy(c2h.at[pl.ds(base, PW_B)], c2v)

    def lk_copy(p, s, start=True):
        cp = (pltpu.async_copy if start else pltpu.make_async_copy)
        return cp(table_h.at[nk.at[p, s]], jb.at[p, s], sls[p])

    def g_copy(p, s, start=True):
        cp = (pltpu.async_copy if start else pltpu.make_async_copy)
        return cp(featsp_h.at[nk.at[p, s]], rows.at[p, s], sgs[p])

    def w_copy(p, s, c, start=True):
        cp = (pltpu.async_copy if start else pltpu.make_async_copy)
        return cp(rows.at[p, s], outs[s].at[pl.ds(base + c * 128, 128), :],
                  sws[p])

    def step(t2, _):
        for q in range(2):
            t = t2 * 2 + q
            @pl.when((t >= 2) & (t <= NCH + 1))
            def _():
                for s in range(6):
                    g_copy(q, s, start=False).wait()
                for s in range(6):
                    w_copy(q, s, t - 2)

            @pl.when(t < NCH)
            def _():
                for g in range(8):
                    off = t * 128 + g * 16
                    c0 = c0v[pl.ds(off, 16)]
                    c1 = c1v[pl.ds(off, 16)]
                    c2 = c2v[pl.ds(off, 16)]
                    cs = (c0, c1, c2)
                    dims = (D0, D1, D2)
                    strides = (S0, S1, 1)
                    key = _keys16(c0, c1, c2)
                    ig = base + off + lax.broadcasted_iota(jnp.int32,
                                                           (16,), 0)
                    zb[q, pl.ds(g * 16, 16)] = N + lax.rem(ig, ZROWS)
                    snt = TOFF + SNT_BASE + ((ig & 1023) << 4)
                    for s, (ax, d) in enumerate(_TAPS):
                        ca = cs[ax] + d
                        inb = (ca >= 0) & (ca < dims[ax])
                        nk[q, s, pl.ds(g * 16, 16)] = jnp.where(
                            inb, TOFF + key + d * strides[ax], snt)
                for s in range(6):
                    lk_copy(q, s)

            p = 1 - q

            @pl.when((t >= 1) & (t <= NCH))
            def _():
                for s in range(6):
                    lk_copy(p, s, start=False).wait()
                for s in range(6):
                    for g in range(8):
                        j = jb[p, s, pl.ds(g * 16, 16)]
                        nk[p, s, pl.ds(g * 16, 16)] = jnp.where(
                            j < 0, zb[p, pl.ds(g * 16, 16)], j)

                @pl.when(t >= 3)
                def _():
                    for s in range(6):
                        w_copy(p, s, 0, start=False).wait()

                for s in range(6):
                    g_copy(p, s)
        return 0

    lax.fori_loop(0, (NCH + 2) // 2, step, 0)
    for p in range(2):
        for s in range(6):
            w_copy(p, s, 0, start=False).wait()


def _gather_taps(c0f, c1f, c2f, table, feats_p):
    mesh = plsc.VectorSubcoreMesh(core_axis_name="c", subcore_axis_name="s")
    out = jax.ShapeDtypeStruct((NPAD, C), jnp.float32)
    return pl.kernel(
        _gather_body,
        out_type=(out,) * 6,
        mesh=mesh,
        compiler_params=pltpu.CompilerParams(use_tc_tiling_on_sc=False),
        scratch_types=[
            pltpu.VMEM((PW_B,), jnp.int32),
            pltpu.VMEM((PW_B,), jnp.int32),
            pltpu.VMEM((PW_B,), jnp.int32),
            pltpu.VMEM((2, 6, 128), jnp.int32),
            pltpu.VMEM((2, 6, 128), jnp.int32),
            pltpu.VMEM((2, 128), jnp.int32),
            pltpu.VMEM((2, 6, 128, C), jnp.float32),
            pltpu.SemaphoreType.DMA,
            pltpu.SemaphoreType.DMA,
            pltpu.SemaphoreType.DMA,
            pltpu.SemaphoreType.DMA,
            pltpu.SemaphoreType.DMA,
            pltpu.SemaphoreType.DMA,
        ],
    )(c0f, c1f, c2f, table, feats_p)


def _conv_stats_kernel(f_ref, g0, g1, g2, g3, g4, g5, w1, w2, w3,
                       r1, r2, r3, s1, s2, s3, q1, q2, q3,
                       acc_s, acc_q):
    i = pl.program_id(0)

    @pl.when(i == 0)
    def _():
        acc_s[...] = jnp.zeros_like(acc_s)
        acc_q[...] = jnp.zeros_like(acc_q)

    gms = (g0, g2, g4)
    gps = (g1, g3, g5)
    ws = (w1, w2, w3)
    rs = (r1, r2, r3)
    f = f_ref[...]
    row = lax.broadcasted_iota(jnp.int32, (BLK, 1), 0) + i * BLK
    validf = jnp.where(row < NVALID4, 1.0, 0.0).astype(jnp.float32)
    for a in range(3):
        w = ws[a]
        raw = (jnp.dot(gms[a][...], w[0], preferred_element_type=jnp.float32)
               + jnp.dot(f, w[1], preferred_element_type=jnp.float32)
               + jnp.dot(gps[a][...], w[2], preferred_element_type=jnp.float32))
        rs[a][...] = raw
        rm = raw * validf
        acc_s[:, a * 128:(a + 1) * 128] += jnp.sum(
            rm.reshape(BLK // 8, 8, 128), axis=0)
        acc_q[:, a * 128:(a + 1) * 128] += jnp.sum(
            (rm * rm).reshape(BLK // 8, 8, 128), axis=0)

    @pl.when(i == NBLK - 1)
    def _():
        s1[...] = acc_s[:, 0:128]
        s2[...] = acc_s[:, 128:256]
        s3[...] = acc_s[:, 256:384]
        q1[...] = acc_q[:, 0:128]
        q2[...] = acc_q[:, 128:256]
        q3[...] = acc_q[:, 256:384]


def _conv_stats(feats4, gath4, wbd):
    blk = pl.BlockSpec((BLK, 128), lambda i: (i, 0))
    wspec = pl.BlockSpec((3, 128, 128), lambda i: (0, 0, 0))
    sspec = pl.BlockSpec((8, 128), lambda i: (0, 0))
    big = jax.ShapeDtypeStruct((NR, 128), jnp.float32)
    small = jax.ShapeDtypeStruct((8, 128), jnp.float32)
    return pl.pallas_call(
        _conv_stats_kernel,
        grid=(NBLK,),
        in_specs=[blk] * 7 + [wspec] * 3,
        out_specs=[blk] * 3 + [sspec] * 6,
        out_shape=[big] * 3 + [small] * 6,
        scratch_shapes=[pltpu.VMEM((8, 384), jnp.float32),
                        pltpu.VMEM((8, 384), jnp.float32)],
    )(feats4, *gath4, *wbd)


def _apply_kernel(f_ref, r1, r2, r3, sc_ref, out_ref):
    f = f_ref[...]
    rs = (r1, r2, r3)
    tot = jnp.zeros((BLK, 128), jnp.float32)
    for a in range(3):
        x = rs[a][...] * sc_ref[a:a + 1, :] + sc_ref[a + 3:a + 4, :]
        tot = tot + 1.0 / (1.0 + jnp.exp(-x))
    out_ref[...] = tot * f


def _apply(feats4, raws, scsh):
    blk = pl.BlockSpec((BLK, 128), lambda i: (i, 0))
    sspec = pl.BlockSpec((8, 128), lambda i: (0, 0))
    return pl.pallas_call(
        _apply_kernel,
        grid=(NBLK,),
        in_specs=[blk] * 4 + [sspec],
        out_specs=blk,
        out_shape=jax.ShapeDtypeStruct((NR, 128), jnp.float32),
    )(feats4, *raws, scsh)


def kernel(feats, coords, W1, W2, W3, g1, b1, g2, b2, g3, b3):
    r = jnp.arange(ZROWS, dtype=jnp.int32)
    c0f = jnp.concatenate([coords[:, 0], jnp.full((ZROWS,), D0, jnp.int32)])
    c1f = jnp.concatenate([coords[:, 1], r // 32])
    c2f = jnp.concatenate([coords[:, 2], r % 32])
    feats_p = jnp.zeros((NPAD, C), jnp.float32).at[:N].set(feats)

    table = _build_table(c0f, c1f, c2f)
    gath = _gather_taps(c0f, c1f, c2f, table, feats_p)

    feats4 = feats_p.reshape(NR, 128)
    gath4 = [g.reshape(NR, 128) for g in gath]

    eye4 = jnp.eye(4, dtype=jnp.float32)
    wbd = [jnp.einsum("kab,ij->kiajb", W, eye4).reshape(3, 128, 128)
           for W in (W1, W2, W3)]
    res = _conv_stats(feats4, gath4, wbd)
    raws = res[0:3]
    sums = res[3:6]
    sqs = res[6:9]

    gs = (g1, g2, g3)
    bs = (b1, b2, b3)
    scsh = []
    for a in range(3):
        s = jnp.sum(sums[a], axis=0).reshape(4, 32).sum(axis=0)
        q = jnp.sum(sqs[a], axis=0).reshape(4, 32).sum(axis=0)
        mean = s / N
        var = q / N - mean * mean
        scale = gs[a] / jnp.sqrt(var + 1e-5)
        shift = bs[a] - mean * scale
        scsh.append(jnp.tile(scale, 4))
        scsh.append(jnp.tile(shift, 4))
    scsh = jnp.stack(scsh[0::2] + scsh[1::2] + [jnp.zeros(128)] * 2)
    scsh = scsh.astype(jnp.float32)

    out4 = _apply(feats4, raws, scsh)
    return out4.reshape(NPAD, C)[:N]


# speedup vs baseline: 1.8363x; 1.2358x over previous
import functools

import jax
import jax.numpy as jnp
from jax import lax
from jax.experimental import pallas as pl
from jax.experimental.pallas import tpu as pltpu
from jax.experimental.pallas import tpu_sc as plsc

D0, D1, D2 = 480, 360, 32
S0, S1 = D1 * D2, D2
N = 200000
C = 32

NPAD = 204800
ZROWS = NPAD - N
TSIZE = D0 * D1 * D2
SNT_BASE = TSIZE + 8192
TTOT = 5767168

NW_B = 32
PW_B = NPAD // NW_B
NW_A = 16
PW_A = NPAD // NW_A
TW_A = TTOT // NW_A
MSET_CH = 32768
NMSET = TW_A // MSET_CH

NR = NPAD // 4
BLK = 512
NBLK = NR // BLK
NVALID4 = N // 4


def _keys16(c0, c1, c2):
    return c0 * S0 + c1 * S1 + c2


TBLK = 262144


def _tmemset_kernel(out_ref):
    out_ref[...] = jnp.full((8, TBLK // 8), -1, jnp.int32)


def _table_memset():
    return pl.pallas_call(
        _tmemset_kernel,
        grid=(TTOT // TBLK,),
        out_specs=pl.BlockSpec((8, TBLK // 8), lambda i: (i, 0)),
        out_shape=jax.ShapeDtypeStruct((TTOT // (TBLK // 8), TBLK // 8),
                                       jnp.int32),
    )()


def _scatter_body(c0h, c1h, c2h, table_h, done_h, c0v, c1v, c2v,
                  keyb0, keyb1, valb0, valb1, sems0, sems1):
    wid = lax.axis_index("s") * 2 + lax.axis_index("c")
    keybs = (keyb0, keyb1)
    valbs = (valb0, valb1)
    semss = (sems0, sems1)
    base = wid * PW_B
    pltpu.sync_copy(c0h.at[pl.ds(base, PW_B)], c0v)
    pltpu.sync_copy(c1h.at[pl.ds(base, PW_B)], c1v)
    pltpu.sync_copy(c2h.at[pl.ds(base, PW_B)], c2v)

    def scatter(t2, carry):
        for q in range(2):
            c = t2 * 2 + q

            @pl.when(c >= 2)
            def _():
                pltpu.make_async_copy(valbs[q], table_h.at[keybs[q]],
                                      semss[q]).wait()

            for g in range(8):
                off = c * 128 + g * 16
                key = _keys16(c0v[pl.ds(off, 16)], c1v[pl.ds(off, 16)],
                              c2v[pl.ds(off, 16)])
                keybs[q][pl.ds(g * 16, 16)] = key
                valbs[q][pl.ds(g * 16, 16)] = (
                    base + off + lax.broadcasted_iota(jnp.int32, (16,), 0))
            pltpu.async_copy(valbs[q], table_h.at[keybs[q]], semss[q])
        return carry

    lax.fori_loop(0, PW_B // 256, scatter, 0)
    for q in range(2):
        pltpu.make_async_copy(valbs[q], table_h.at[keybs[q]], semss[q]).wait()
    keyb0[pl.ds(0, 16)] = jnp.full((16,), 1, jnp.int32)
    pltpu.sync_copy(keyb0.at[pl.ds(0, 16)], done_h.at[pl.ds(0, 16)])


def _build_table(c0f, c1f, c2f):
    table0 = _table_memset().reshape(TTOT)
    mesh = plsc.VectorSubcoreMesh(core_axis_name="c", subcore_axis_name="s")
    done = pl.kernel(
        _scatter_body,
        out_type=jax.ShapeDtypeStruct((16,), jnp.int32),
        mesh=mesh,
        compiler_params=pltpu.CompilerParams(
            use_tc_tiling_on_sc=False, has_side_effects=True),
        scratch_types=[
            pltpu.VMEM((PW_B,), jnp.int32),
            pltpu.VMEM((PW_B,), jnp.int32),
            pltpu.VMEM((PW_B,), jnp.int32),
            pltpu.VMEM((128,), jnp.int32),
            pltpu.VMEM((128,), jnp.int32),
            pltpu.VMEM((128,), jnp.int32),
            pltpu.VMEM((128,), jnp.int32),
            pltpu.SemaphoreType.DMA,
            pltpu.SemaphoreType.DMA,
        ],
    )(c0f, c1f, c2f, table0)
    return table0, done


_TAPS = ((0, -1), (0, 1), (1, -1), (1, 1), (2, -1), (2, 1))


NCH = PW_B // 128


def _gather_body(c0h, c1h, c2h, table_h, done_h, featsp_h,
                 o0, o1, o2, o3, o4, o5,
                 c0v, c1v, c2v, nk, jb, zb, rows,
                 sl0, sl1, sg0, sg1, sw0, sw1):
    outs = (o0, o1, o2, o3, o4, o5)
    sls = (sl0, sl1)
    sgs = (sg0, sg1)
    sws = (sw0, sw1)
    wid = lax.axis_index("s") * 2 + lax.axis_index("c")
    base = wid * PW_B
    pltpu.sync_copy(c0h.at[pl.ds(base, PW_B)], c0v)
    pltpu.sync_copy(c1h.at[pl.ds(base, PW_B)], c1v)
    pltpu.sync_copy(c2h.at[pl.ds(base, PW_B)], c2v)

    def lk_copy(p, s, start=True):
        cp = (pltpu.async_copy if start else pltpu.make_async_copy)
        return cp(table_h.at[nk.at[p, s]], jb.at[p, s], sls[p])

    def g_copy(p, s, start=True):
        cp = (pltpu.async_copy if start else pltpu.make_async_copy)
        return cp(featsp_h.at[nk.at[p, s]], rows.at[p, s], sgs[p])

    def w_copy(p, s, c, start=True):
        cp = (pltpu.async_copy if start else pltpu.make_async_copy)
        return cp(rows.at[p, s], outs[s].at[pl.ds(base + c * 128, 128), :],
                  sws[p])

    def step(t2, _):
        for q in range(2):
            t = t2 * 2 + q
            @pl.when((t >= 2) & (t <= NCH + 1))
            def _():
                for s in range(6):
                    g_copy(q, s, start=False).wait()
                for s in range(6):
                    w_copy(q, s, t - 2)

            @pl.when(t < NCH)
            def _():
                for g in range(8):
                    off = t * 128 + g * 16
                    c0 = c0v[pl.ds(off, 16)]
                    c1 = c1v[pl.ds(off, 16)]
                    c2 = c2v[pl.ds(off, 16)]
                    cs = (c0, c1, c2)
                    dims = (D0, D1, D2)
                    strides = (S0, S1, 1)
                    key = _keys16(c0, c1, c2)
                    ig = base + off + lax.broadcasted_iota(jnp.int32,
                                                           (16,), 0)
                    zb[q, pl.ds(g * 16, 16)] = N + lax.rem(ig, ZROWS)
                    snt = SNT_BASE + ((ig & 1023) << 4)
                    for s, (ax, d) in enumerate(_TAPS):
                        ca = cs[ax] + d
                        inb = (ca >= 0) & (ca < dims[ax])
                        nk[q, s, pl.ds(g * 16, 16)] = jnp.where(
                            inb, key + d * strides[ax], snt)
                for s in range(6):
                    lk_copy(q, s)

            p = 1 - q

            @pl.when((t >= 1) & (t <= NCH))
            def _():
                for s in range(6):
                    lk_copy(p, s, start=False).wait()
                for s in range(6):
                    for g in range(8):
                        j = jb[p, s, pl.ds(g * 16, 16)]
                        nk[p, s, pl.ds(g * 16, 16)] = jnp.where(
                            j < 0, zb[p, pl.ds(g * 16, 16)], j)

                @pl.when(t >= 3)
                def _():
                    for s in range(6):
                        w_copy(p, s, 0, start=False).wait()

                for s in range(6):
                    g_copy(p, s)
        return 0

    lax.fori_loop(0, (NCH + 2) // 2, step, 0)
    for p in range(2):
        for s in range(6):
            w_copy(p, s, 0, start=False).wait()


def _gather_taps(c0f, c1f, c2f, table, done, feats_p):
    mesh = plsc.VectorSubcoreMesh(core_axis_name="c", subcore_axis_name="s")
    out = jax.ShapeDtypeStruct((NPAD, C), jnp.float32)
    return pl.kernel(
        _gather_body,
        out_type=(out,) * 6,
        mesh=mesh,
        compiler_params=pltpu.CompilerParams(use_tc_tiling_on_sc=False),
        scratch_types=[
            pltpu.VMEM((PW_B,), jnp.int32),
            pltpu.VMEM((PW_B,), jnp.int32),
            pltpu.VMEM((PW_B,), jnp.int32),
            pltpu.VMEM((2, 6, 128), jnp.int32),
            pltpu.VMEM((2, 6, 128), jnp.int32),
            pltpu.VMEM((2, 128), jnp.int32),
            pltpu.VMEM((2, 6, 128, C), jnp.float32),
            pltpu.SemaphoreType.DMA,
            pltpu.SemaphoreType.DMA,
            pltpu.SemaphoreType.DMA,
            pltpu.SemaphoreType.DMA,
            pltpu.SemaphoreType.DMA,
            pltpu.SemaphoreType.DMA,
        ],
    )(c0f, c1f, c2f, table, done, feats_p)


def _conv_stats_kernel(f_ref, g0, g1, g2, g3, g4, g5, w1, w2, w3,
                       r1, r2, r3, s1, s2, s3, q1, q2, q3,
                       acc_s, acc_q):
    i = pl.program_id(0)

    @pl.when(i == 0)
    def _():
        acc_s[...] = jnp.zeros_like(acc_s)
        acc_q[...] = jnp.zeros_like(acc_q)

    gms = (g0, g2, g4)
    gps = (g1, g3, g5)
    ws = (w1, w2, w3)
    rs = (r1, r2, r3)
    f = f_ref[...]
    row = lax.broadcasted_iota(jnp.int32, (BLK, 1), 0) + i * BLK
    validf = jnp.where(row < NVALID4, 1.0, 0.0).astype(jnp.float32)
    for a in range(3):
        w = ws[a]
        raw = (jnp.dot(gms[a][...], w[0], preferred_element_type=jnp.float32)
               + jnp.dot(f, w[1], preferred_element_type=jnp.float32)
               + jnp.dot(gps[a][...], w[2], preferred_element_type=jnp.float32))
        rs[a][...] = raw
        rm = raw * validf
        acc_s[:, a * 128:(a + 1) * 128] += jnp.sum(
            rm.reshape(BLK // 8, 8, 128), axis=0)
        acc_q[:, a * 128:(a + 1) * 128] += jnp.sum(
            (rm * rm).reshape(BLK // 8, 8, 128), axis=0)

    @pl.when(i == NBLK - 1)
    def _():
        s1[...] = acc_s[:, 0:128]
        s2[...] = acc_s[:, 128:256]
        s3[...] = acc_s[:, 256:384]
        q1[...] = acc_q[:, 0:128]
        q2[...] = acc_q[:, 128:256]
        q3[...] = acc_q[:, 256:384]


def _conv_stats(feats4, gath4, wbd):
    blk = pl.BlockSpec((BLK, 128), lambda i: (i, 0))
    wspec = pl.BlockSpec((3, 128, 128), lambda i: (0, 0, 0))
    sspec = pl.BlockSpec((8, 128), lambda i: (0, 0))
    big = jax.ShapeDtypeStruct((NR, 128), jnp.float32)
    small = jax.ShapeDtypeStruct((8, 128), jnp.float32)
    return pl.pallas_call(
        _conv_stats_kernel,
        grid=(NBLK,),
        in_specs=[blk] * 7 + [wspec] * 3,
        out_specs=[blk] * 3 + [sspec] * 6,
        out_shape=[big] * 3 + [small] * 6,
        scratch_shapes=[pltpu.VMEM((8, 384), jnp.float32),
                        pltpu.VMEM((8, 384), jnp.float32)],
    )(feats4, *gath4, *wbd)


def _apply_kernel(f_ref, r1, r2, r3, sc_ref, out_ref):
    f = f_ref[...]
    rs = (r1, r2, r3)
    tot = jnp.zeros((BLK, 128), jnp.float32)
    for a in range(3):
        x = rs[a][...] * sc_ref[a:a + 1, :] + sc_ref[a + 3:a + 4, :]
        tot = tot + 1.0 / (1.0 + jnp.exp(-x))
    out_ref[...] = tot * f


def _apply(feats4, raws, scsh):
    blk = pl.BlockSpec((BLK, 128), lambda i: (i, 0))
    sspec = pl.BlockSpec((8, 128), lambda i: (0, 0))
    return pl.pallas_call(
        _apply_kernel,
        grid=(NBLK,),
        in_specs=[blk] * 4 + [sspec],
        out_specs=blk,
        out_shape=jax.ShapeDtypeStruct((NR, 128), jnp.float32),
    )(feats4, *raws, scsh)


def kernel(feats, coords, W1, W2, W3, g1, b1, g2, b2, g3, b3):
    r = jnp.arange(ZROWS, dtype=jnp.int32)
    c0f = jnp.concatenate([coords[:, 0], jnp.full((ZROWS,), D0, jnp.int32)])
    c1f = jnp.concatenate([coords[:, 1], r // 32])
    c2f = jnp.concatenate([coords[:, 2], r % 32])
    feats_p = jnp.zeros((NPAD, C), jnp.float32).at[:N].set(feats)

    table, done = _build_table(c0f, c1f, c2f)
    gath = _gather_taps(c0f, c1f, c2f, table, done, feats_p)

    feats4 = feats_p.reshape(NR, 128)
    gath4 = [g.reshape(NR, 128) for g in gath]

    eye4 = jnp.eye(4, dtype=jnp.float32)
    wbd = [jnp.einsum("kab,ij->kiajb", W, eye4).reshape(3, 128, 128)
           for W in (W1, W2, W3)]
    res = _conv_stats(feats4, gath4, wbd)
    raws = res[0:3]
    sums = res[3:6]
    sqs = res[6:9]

    gs = (g1, g2, g3)
    bs = (b1, b2, b3)
    scsh = []
    for a in range(3):
        s = jnp.sum(sums[a], axis=0).reshape(4, 32).sum(axis=0)
        q = jnp.sum(sqs[a], axis=0).reshape(4, 32).sum(axis=0)
        mean = s / N
        var = q / N - mean * mean
        scale = gs[a] / jnp.sqrt(var + 1e-5)
        shift = bs[a] - mean * scale
        scsh.append(jnp.tile(scale, 4))
        scsh.append(jnp.tile(shift, 4))
    scsh = jnp.stack(scsh[0::2] + scsh[1::2] + [jnp.zeros(128)] * 2)
    scsh = scsh.astype(jnp.float32)

    out4 = _apply(feats4, raws, scsh)
    return out4.reshape(NPAD, C)[:N]


# speedup vs baseline: 2.2836x; 1.2436x over previous
import functools

import jax
import jax.numpy as jnp
from jax import lax
from jax.experimental import pallas as pl
from jax.experimental.pallas import tpu as pltpu
from jax.experimental.pallas import tpu_sc as plsc

D0, D1, D2 = 480, 360, 32
S0, S1 = D1 * D2, D2
N = 200000
C = 32

NPAD = 204800
ZROWS = NPAD - N
TSIZE = D0 * D1 * D2
SNT_BASE = TSIZE + 8192
TTOT = 5767168

NW_B = 32
PW_B = NPAD // NW_B
HHALF = TTOT // 2
PW_A = NPAD // 16

NR = NPAD // 4
BLK = 512
NBLK = NR // BLK
NVALID4 = N // 4


def _keys16(c0, c1, c2):
    return c0 * S0 + c1 * S1 + c2


ROUNDS = ((0, 983040), (983040, 983040), (1966080, 917504))
BUFW = 1179648
MS_CH = 8192


def _table_build_body(c0h, c1h, c2h, table_h, mset, c0v, c1v, c2v,
                      keyb0, keyb1, valb0, valb1, shared,
                      semm, sems0, sems1, semo):
    ci = lax.axis_index("c")
    si = lax.axis_index("s")
    keybs = (keyb0, keyb1)
    valbs = (valb0, valb1)
    semss = (sems0, sems1)

    def fill(i, carry):
        mset[pl.ds(i * 16, 16)] = jnp.full((16,), -1, jnp.int32)
        return carry

    lax.fori_loop(0, MS_CH // 16, fill, 0)

    base = si * PW_A
    pltpu.sync_copy(c0h.at[pl.ds(base, PW_A)], c0v)
    pltpu.sync_copy(c1h.at[pl.ds(base, PW_A)], c1v)
    pltpu.sync_copy(c2h.at[pl.ds(base, PW_A)], c2v)

    for off_r, rw in ROUNDS:
        rbase = ci * HHALF + off_r
        mbase = si * (BUFW // 16)

        def memset(i, carry):
            pltpu.async_copy(mset, shared.at[pl.ds(mbase + i * MS_CH,
                                                   MS_CH)], semm)
            return carry

        lax.fori_loop(0, BUFW // 16 // MS_CH, memset, 0)

        def memdrain(i, carry):
            pltpu.make_async_copy(mset, shared.at[pl.ds(mbase, MS_CH)],
                                  semm).wait()
            return carry

        lax.fori_loop(0, BUFW // 16 // MS_CH, memdrain, 0)
        plsc.subcore_barrier()

        def scatter(t2, carry):
            for q in range(2):
                c = t2 * 2 + q

                @pl.when(c >= 2)
                def _():
                    pltpu.make_async_copy(valbs[q], shared.at[keybs[q]],
                                          semss[q]).wait()

                for g in range(8):
                    off = c * 128 + g * 16
                    slot = _keys16(c0v[pl.ds(off, 16)],
                                   c1v[pl.ds(off, 16)],
                                   c2v[pl.ds(off, 16)])
                    ig = base + off + lax.broadcasted_iota(jnp.int32,
                                                           (16,), 0)
                    mine = (slot >= rbase) & (slot < rbase + rw)
                    trash = 983040 + ((ig & 4095) << 2)
                    keybs[q][pl.ds(g * 16, 16)] = jnp.where(
                        mine, slot - rbase, trash)
                    valbs[q][pl.ds(g * 16, 16)] = ig
                pltpu.async_copy(valbs[q], shared.at[keybs[q]], semss[q])
            return carry

        lax.fori_loop(0, PW_A // 256, scatter, 0)
        for q in range(2):
            pltpu.make_async_copy(valbs[q], shared.at[keybs[q]],
                                  semss[q]).wait()
        plsc.subcore_barrier()
        out_ch = rw // 16
        pltpu.async_copy(shared.at[pl.ds(si * out_ch, out_ch)],
                         table_h.at[pl.ds(rbase + si * out_ch, out_ch)],
                         semo).wait()


def _build_table(c0f, c1f, c2f):
    mesh = plsc.VectorSubcoreMesh(core_axis_name="c", subcore_axis_name="s")
    return pl.kernel(
        _table_build_body,
        out_type=jax.ShapeDtypeStruct((TTOT,), jnp.int32),
        mesh=mesh,
        compiler_params=pltpu.CompilerParams(use_tc_tiling_on_sc=False),
        scratch_types=[
            pltpu.VMEM((MS_CH,), jnp.int32),
            pltpu.VMEM((PW_A,), jnp.int32),
            pltpu.VMEM((PW_A,), jnp.int32),
            pltpu.VMEM((PW_A,), jnp.int32),
            pltpu.VMEM((128,), jnp.int32),
            pltpu.VMEM((128,), jnp.int32),
            pltpu.VMEM((128,), jnp.int32),
            pltpu.VMEM((128,), jnp.int32),
            pltpu.VMEM_SHARED((BUFW,), jnp.int32),
            pltpu.SemaphoreType.DMA,
            pltpu.SemaphoreType.DMA,
            pltpu.SemaphoreType.DMA,
            pltpu.SemaphoreType.DMA,
        ],
    )(c0f, c1f, c2f)


_TAPS = ((0, -1), (0, 1), (1, -1), (1, 1), (2, -1), (2, 1))


NCH = PW_B // 128


def _gather_body(c0h, c1h, c2h, table_h, featsp_h,
                 o0, o1, o2, o3, o4, o5,
                 c0v, c1v, c2v, nk, jb, zb, rows,
                 sl0, sl1, sg0, sg1, sw0, sw1):
    outs = (o0, o1, o2, o3, o4, o5)
    sls = (sl0, sl1)
    sgs = (sg0, sg1)
    sws = (sw0, sw1)
    wid = lax.axis_index("s") * 2 + lax.axis_index("c")
    base = wid * PW_B
    pltpu.sync_copy(c0h.at[pl.ds(base, PW_B)], c0v)
    pltpu.sync_copy(c1h.at[pl.ds(base, PW_B)], c1v)
    pltpu.sync_copy(c2h.at[pl.ds(base, PW_B)], c2v)

    def lk_copy(p, s, start=True):
        cp = (pltpu.async_copy if start else pltpu.make_async_copy)
        return cp(table_h.at[nk.at[p, s]], jb.at[p, s], sls[p])

    def g_copy(p, s, start=True):
        cp = (pltpu.async_copy if start else pltpu.make_async_copy)
        return cp(featsp_h.at[nk.at[p, s]], rows.at[p, s], sgs[p])

    def w_copy(p, s, c, start=True):
        cp = (pltpu.async_copy if start else pltpu.make_async_copy)
        return cp(rows.at[p, s], outs[s].at[pl.ds(base + c * 128, 128), :],
                  sws[p])

    def step(t2, _):
        for q in range(2):
            t = t2 * 2 + q
            @pl.when((t >= 2) & (t <= NCH + 1))
            def _():
                for s in range(6):
                    g_copy(q, s, start=False).wait()
                for s in range(6):
                    w_copy(q, s, t - 2)

            @pl.when(t < NCH)
            def _():
                for g in range(8):
                    off = t * 128 + g * 16
                    c0 = c0v[pl.ds(off, 16)]
                    c1 = c1v[pl.ds(off, 16)]
                    c2 = c2v[pl.ds(off, 16)]
                    cs = (c0, c1, c2)
                    dims = (D0, D1, D2)
                    strides = (S0, S1, 1)
                    key = _keys16(c0, c1, c2)
                    ig = base + off + lax.broadcasted_iota(jnp.int32,
                                                           (16,), 0)
                    zb[q, pl.ds(g * 16, 16)] = N + lax.rem(ig, ZROWS)
                    snt = SNT_BASE + ((ig & 1023) << 4)
                    for s, (ax, d) in enumerate(_TAPS):
                        ca = cs[ax] + d
                        inb = (ca >= 0) & (ca < dims[ax])
                        nk[q, s, pl.ds(g * 16, 16)] = jnp.where(
                            inb, key + d * strides[ax], snt)
                for s in range(6):
                    lk_copy(q, s)

            p = 1 - q

            @pl.when((t >= 1) & (t <= NCH))
            def _():
                for s in range(6):
                    lk_copy(p, s, start=False).wait()
                for s in range(6):
                    for g in range(8):
                        j = jb[p, s, pl.ds(g * 16, 16)]
                        nk[p, s, pl.ds(g * 16, 16)] = jnp.where(
                            j < 0, zb[p, pl.ds(g * 16, 16)], j)

                @pl.when(t >= 3)
                def _():
                    for s in range(6):
                        w_copy(p, s, 0, start=False).wait()

                for s in range(6):
                    g_copy(p, s)
        return 0

    lax.fori_loop(0, (NCH + 2) // 2, step, 0)
    for p in range(2):
        for s in range(6):
            w_copy(p, s, 0, start=False).wait()


def _gather_taps(c0f, c1f, c2f, table, feats_p):
    mesh = plsc.VectorSubcoreMesh(core_axis_name="c", subcore_axis_name="s")
    out = jax.ShapeDtypeStruct((NPAD, C), jnp.float32)
    return pl.kernel(
        _gather_body,
        out_type=(out,) * 6,
        mesh=mesh,
        compiler_params=pltpu.CompilerParams(use_tc_tiling_on_sc=False),
        scratch_types=[
            pltpu.VMEM((PW_B,), jnp.int32),
            pltpu.VMEM((PW_B,), jnp.int32),
            pltpu.VMEM((PW_B,), jnp.int32),
            pltpu.VMEM((2, 6, 128), jnp.int32),
            pltpu.VMEM((2, 6, 128), jnp.int32),
            pltpu.VMEM((2, 128), jnp.int32),
            pltpu.VMEM((2, 6, 128, C), jnp.float32),
            pltpu.SemaphoreType.DMA,
            pltpu.SemaphoreType.DMA,
            pltpu.SemaphoreType.DMA,
            pltpu.SemaphoreType.DMA,
            pltpu.SemaphoreType.DMA,
            pltpu.SemaphoreType.DMA,
        ],
    )(c0f, c1f, c2f, table, feats_p)


def _conv_stats_kernel(f_ref, g0, g1, g2, g3, g4, g5, w1, w2, w3,
                       r1, r2, r3, s1, s2, s3, q1, q2, q3,
                       acc_s, acc_q):
    i = pl.program_id(0)

    @pl.when(i == 0)
    def _():
        acc_s[...] = jnp.zeros_like(acc_s)
        acc_q[...] = jnp.zeros_like(acc_q)

    gms = (g0, g2, g4)
    gps = (g1, g3, g5)
    ws = (w1, w2, w3)
    rs = (r1, r2, r3)
    f = f_ref[...]
    row = lax.broadcasted_iota(jnp.int32, (BLK, 1), 0) + i * BLK
    validf = jnp.where(row < NVALID4, 1.0, 0.0).astype(jnp.float32)
    for a in range(3):
        w = ws[a]
        raw = (jnp.dot(gms[a][...], w[0], preferred_element_type=jnp.float32)
               + jnp.dot(f, w[1], preferred_element_type=jnp.float32)
               + jnp.dot(gps[a][...], w[2], preferred_element_type=jnp.float32))
        rs[a][...] = raw
        rm = raw * validf
        acc_s[:, a * 128:(a + 1) * 128] += jnp.sum(
            rm.reshape(BLK // 8, 8, 128), axis=0)
        acc_q[:, a * 128:(a + 1) * 128] += jnp.sum(
            (rm * rm).reshape(BLK // 8, 8, 128), axis=0)

    @pl.when(i == NBLK - 1)
    def _():
        s1[...] = acc_s[:, 0:128]
        s2[...] = acc_s[:, 128:256]
        s3[...] = acc_s[:, 256:384]
        q1[...] = acc_q[:, 0:128]
        q2[...] = acc_q[:, 128:256]
        q3[...] = acc_q[:, 256:384]


def _conv_stats(feats4, gath4, wbd):
    blk = pl.BlockSpec((BLK, 128), lambda i: (i, 0))
    wspec = pl.BlockSpec((3, 128, 128), lambda i: (0, 0, 0))
    sspec = pl.BlockSpec((8, 128), lambda i: (0, 0))
    big = jax.ShapeDtypeStruct((NR, 128), jnp.float32)
    small = jax.ShapeDtypeStruct((8, 128), jnp.float32)
    return pl.pallas_call(
        _conv_stats_kernel,
        grid=(NBLK,),
        in_specs=[blk] * 7 + [wspec] * 3,
        out_specs=[blk] * 3 + [sspec] * 6,
        out_shape=[big] * 3 + [small] * 6,
        scratch_shapes=[pltpu.VMEM((8, 384), jnp.float32),
                        pltpu.VMEM((8, 384), jnp.float32)],
    )(feats4, *gath4, *wbd)


def _apply_kernel(f_ref, r1, r2, r3, sc_ref, out_ref):
    f = f_ref[...]
    rs = (r1, r2, r3)
    tot = jnp.zeros((BLK, 128), jnp.float32)
    for a in range(3):
        x = rs[a][...] * sc_ref[a:a + 1, :] + sc_ref[a + 3:a + 4, :]
        tot = tot + 1.0 / (1.0 + jnp.exp(-x))
    out_ref[...] = tot * f


def _apply(feats4, raws, scsh):
    blk = pl.BlockSpec((BLK, 128), lambda i: (i, 0))
    sspec = pl.BlockSpec((8, 128), lambda i: (0, 0))
    return pl.pallas_call(
        _apply_kernel,
        grid=(NBLK,),
        in_specs=[blk] * 4 + [sspec],
        out_specs=blk,
        out_shape=jax.ShapeDtypeStruct((NR, 128), jnp.float32),
    )(feats4, *raws, scsh)


def kernel(feats, coords, W1, W2, W3, g1, b1, g2, b2, g3, b3):
    r = jnp.arange(ZROWS, dtype=jnp.int32)
    c0f = jnp.concatenate([coords[:, 0], jnp.full((ZROWS,), D0, jnp.int32)])
    c1f = jnp.concatenate([coords[:, 1], r // 32])
    c2f = jnp.concatenate([coords[:, 2], r % 32])
    feats_p = jnp.zeros((NPAD, C), jnp.float32).at[:N].set(feats)

    table = _build_table(c0f, c1f, c2f)
    gath = _gather_taps(c0f, c1f, c2f, table, feats_p)

    feats4 = feats_p.reshape(NR, 128)
    gath4 = [g.reshape(NR, 128) for g in gath]

    eye4 = jnp.eye(4, dtype=jnp.float32)
    wbd = [jnp.einsum("kab,ij->kiajb", W, eye4).reshape(3, 128, 128)
           for W in (W1, W2, W3)]
    res = _conv_stats(feats4, gath4, wbd)
    raws = res[0:3]
    sums = res[3:6]
    sqs = res[6:9]

    gs = (g1, g2, g3)
    bs = (b1, b2, b3)
    scsh = []
    for a in range(3):
        s = jnp.sum(sums[a], axis=0).reshape(4, 32).sum(axis=0)
        q = jnp.sum(sqs[a], axis=0).reshape(4, 32).sum(axis=0)
        mean = s / N
        var = q / N - mean * mean
        scale = gs[a] / jnp.sqrt(var + 1e-5)
        shift = bs[a] - mean * scale
        scsh.append(jnp.tile(scale, 4))
        scsh.append(jnp.tile(shift, 4))
    scsh = jnp.stack(scsh[0::2] + scsh[1::2] + [jnp.zeros(128)] * 2)
    scsh = scsh.astype(jnp.float32)

    out4 = _apply(feats4, raws, scsh)
    return out4.reshape(NPAD, C)[:N]
